# baseline (speedup 1.0000x reference)
import functools
import math

import jax
import jax.numpy as jnp
from jax import lax
from jax.experimental import pallas as pl
from jax.experimental.pallas import tpu as pltpu
from jax.experimental.pallas import tpu_sc as plsc

NUM_LAYERS = 4
ALPHA = 0.1
THETA = 0.5
NUM_GRAPHS = 64

_NC = 2
_NS = 16
_NW = _NC * _NS
_CH = 128
_BT = 256


def _sc_mesh():
    return plsc.VectorSubcoreMesh(core_axis_name="c", subcore_axis_name="s")


def _deg_kernel(npad, kd):
    @functools.partial(
        pl.kernel,
        out_type=jax.ShapeDtypeStruct((_NC, npad), jnp.float32),
        mesh=_sc_mesh(),
        scratch_types=[
            pltpu.VMEM((kd, _CH), jnp.int32),
            pltpu.VMEM((_CH,), jnp.float32),
            pltpu.VMEM_SHARED((npad,), jnp.float32),
        ],
    )
    def k(col_hbm, zeros_hbm, deg_out, cidx, ones_v, deg_sh):
        cid = lax.axis_index("c")
        sid = lax.axis_index("s")
        wid = cid * _NS + sid
        rp = npad // _NS
        pltpu.sync_copy(zeros_hbm.at[pl.ds(sid * rp, rp)],
                        deg_sh.at[pl.ds(sid * rp, rp)])
        pltpu.sync_copy(col_hbm.at[wid], cidx)
        for j in range(_CH // 16):
            ones_v[pl.ds(j * 16, 16)] = jnp.ones((16,), jnp.float32)
        plsc.subcore_barrier()

        def dbody(j, c):
            pltpu.sync_copy(ones_v, deg_sh.at[cidx.at[j]], add=True)
            return c
        lax.fori_loop(0, kd, dbody, 0)

        plsc.subcore_barrier()
        pltpu.sync_copy(deg_sh.at[pl.ds(sid * rp, rp)],
                        deg_out.at[cid, pl.ds(sid * rp, rp)])

    return k


def _scatter_rows_kernel(mpad, k, k_light, kx, light_core):
    @functools.partial(
        pl.kernel,
        out_type=jax.ShapeDtypeStruct((_NC, mpad, 128), jnp.float32),
        mesh=_sc_mesh(),
        scratch_types=[
            pltpu.VMEM((k, _CH), jnp.int32),
            pltpu.VMEM((k, _CH), jnp.int32),
            pltpu.VMEM((2, _CH), jnp.int32),
            pltpu.VMEM((_CH, 128), jnp.float32),
            pltpu.VMEM_SHARED((mpad, 128), jnp.float32),
            pltpu.SemaphoreType.DMA,
        ],
    )
    def kfn(tab_hbm, ridx_hbm, cidx_hbm, xidx_hbm, zeros_hbm, out_hbm,
            ridx, cidx, xbuf, rows_v, agg_sh, sem):
        cid = lax.axis_index("c")
        sid = lax.axis_index("s")
        wid = cid * _NS + sid
        rp = mpad // _NS
        pltpu.sync_copy(zeros_hbm.at[pl.ds(sid * rp, rp)],
                        agg_sh.at[pl.ds(sid * rp, rp)])
        pltpu.sync_copy(ridx_hbm.at[wid], ridx)
        pltpu.sync_copy(cidx_hbm.at[wid], cidx)
        plsc.subcore_barrier()

        kk = jnp.where(cid == light_core, k_light, k)

        def body(j, c):
            pltpu.async_copy(tab_hbm.at[ridx.at[j]], rows_v, sem).wait()
            pltpu.sync_copy(rows_v, agg_sh.at[cidx.at[j]], add=True)
            return c
        lax.fori_loop(0, kk, body, 0)

        @pl.when(cid != light_core)
        def _():
            def xbody(j, c):
                pltpu.sync_copy(xidx_hbm.at[sid, j], xbuf)
                pltpu.async_copy(tab_hbm.at[xbuf.at[0]], rows_v, sem).wait()
                pltpu.sync_copy(rows_v, agg_sh.at[xbuf.at[1]], add=True)
                return c
            lax.fori_loop(0, kx, xbody, 0)

        plsc.subcore_barrier()
        pltpu.sync_copy(agg_sh.at[pl.ds(sid * rp, rp)],
                        out_hbm.at[cid, pl.ds(sid * rp, rp)])

    return kfn


def _lin0_call(xp, w, b, degp, npad):
    nblk = npad // _BT

    def body(x_ref, w_ref, b_ref, deg_ref, h_ref, hs_ref, db_ref):
        d = deg_ref[0, :] + deg_ref[1, :] + 1.0
        dinv = lax.rsqrt(d)
        h = jnp.maximum(
            jnp.dot(x_ref[...], w_ref[...],
                    preferred_element_type=jnp.float32) + b_ref[...], 0.0)
        db = jnp.broadcast_to(dinv[:, None], h.shape)
        h_ref[...] = h
        hs_ref[...] = h * db
        db_ref[...] = db

    o = jax.ShapeDtypeStruct((npad, 128), jnp.float32)
    return pl.pallas_call(
        body,
        grid=(nblk,),
        in_specs=[
            pl.BlockSpec((_BT, 128), lambda i: (i, 0)),
            pl.BlockSpec((128, 128), lambda i: (0, 0)),
            pl.BlockSpec((1, 128), lambda i: (0, 0)),
            pl.BlockSpec((2, _BT), lambda i: (0, i)),
        ],
        out_specs=[pl.BlockSpec((_BT, 128), lambda i: (i, 0))] * 3,
        out_shape=[o, o, o],
    )(xp, w, b, degp)


def _layer_call(p, h, x0, db, w, beta, npad):
    nblk = npad // _BT
    a1 = 1.0 - ALPHA
    b1 = 1.0 - beta

    def body(p_ref, h_ref, x0_ref, db_ref, w_ref, hn_ref, hs_ref):
        dbv = db_ref[...]
        s = p_ref[0] + p_ref[1]
        agg = dbv * s + dbv * dbv * h_ref[...]
        out = a1 * agg + ALPHA * x0_ref[...]
        m = jnp.dot(out, w_ref[...], preferred_element_type=jnp.float32)
        hn = jnp.maximum(b1 * out + beta * m, 0.0)
        hn_ref[...] = hn
        hs_ref[...] = hn * dbv

    o = jax.ShapeDtypeStruct((npad, 128), jnp.float32)
    return pl.pallas_call(
        body,
        grid=(nblk,),
        in_specs=[
            pl.BlockSpec((2, _BT, 128), lambda i: (0, i, 0)),
            pl.BlockSpec((_BT, 128), lambda i: (i, 0)),
            pl.BlockSpec((_BT, 128), lambda i: (i, 0)),
            pl.BlockSpec((_BT, 128), lambda i: (i, 0)),
            pl.BlockSpec((128, 128), lambda i: (0, 0)),
        ],
        out_specs=[pl.BlockSpec((_BT, 128), lambda i: (i, 0))] * 2,
        out_shape=[o, o],
    )(p, h, x0, db, w)


def _pool_final_call(h, bat2, w, b, npad):
    nblk = npad // _BT

    def body(bat_ref, h_ref, w_ref, b_ref, o_ref, psum, pcnt):
        i = pl.program_id(0)

        @pl.when(i == 0)
        def _():
            psum[...] = jnp.zeros_like(psum)
            pcnt[...] = jnp.zeros_like(pcnt)

        oh = jnp.equal(
            bat_ref[0][:, None],
            lax.broadcasted_iota(jnp.int32, (_BT, NUM_GRAPHS), 1)
        ).astype(jnp.float32)
        dn = (((0,), (0,)), ((), ()))
        hv = h_ref[...]
        psum[...] += lax.dot_general(oh, hv, dn,
                                     preferred_element_type=jnp.float32)
        pcnt[...] += lax.dot_general(oh, jnp.ones_like(hv), dn,
                                     preferred_element_type=jnp.float32)

        @pl.when(i == nblk - 1)
        def _():
            pooled = psum[...] / jnp.maximum(pcnt[...], 1.0)
            o_ref[...] = jnp.dot(pooled, w_ref[...],
                                 preferred_element_type=jnp.float32) \
                + b_ref[...]

    return pl.pallas_call(
        body,
        grid=(nblk,),
        in_specs=[
            pl.BlockSpec((1, _BT), lambda i: (0, i)),
            pl.BlockSpec((_BT, 128), lambda i: (i, 0)),
            pl.BlockSpec((128, 128), lambda i: (0, 0)),
            pl.BlockSpec((1, 128), lambda i: (0, 0)),
        ],
        out_specs=pl.BlockSpec((NUM_GRAPHS, 128), lambda i: (0, 0)),
        out_shape=jax.ShapeDtypeStruct((NUM_GRAPHS, 128), jnp.float32),
        scratch_shapes=[
            pltpu.VMEM((NUM_GRAPHS, 128), jnp.float32),
            pltpu.VMEM((NUM_GRAPHS, 128), jnp.float32),
        ],
    )(bat2, h, w, b)


def _ceil_to(v, m):
    return -(-v // m) * m


def kernel(x, edge_index, edge_attr, batch, lin0_w, lin0_b, conv_w,
           lin1_w, lin1_b):
    n = x.shape[0]
    e = edge_index.shape[1]
    npad = _ceil_to(n + 1, 2048)

    ke = _ceil_to(e, _NW * _CH) // (_NW * _CH)
    ep = ke * _NW * _CH

    row = edge_index[0]
    col = edge_index[1]
    colr = jnp.concatenate(
        [col, jnp.full((ep - e,), n, jnp.int32)]).reshape(_NW, ke, _CH)
    bat2 = jnp.concatenate(
        [batch, jnp.full((npad - n,), NUM_GRAPHS, jnp.int32)]).reshape(1, npad)

    light = 1
    kt = _ceil_to(e, _NS * _CH) // (_NS * _CH)
    kl = min(ke, max(0, round(kt * 0.35)))
    kx = max(0, kt - ke - kl)
    cap = _NS * (kl + ke + kx) * _CH
    rowp = jnp.concatenate([row, jnp.zeros((cap - e,), jnp.int32)])
    colp = jnp.concatenate([col, jnp.full((cap - e,), n, jnp.int32)])
    sl, sh = _NS * kl * _CH, _NS * ke * _CH

    def parts(a):
        lp = jnp.concatenate(
            [a[:sl].reshape(_NS, kl, _CH),
             jnp.zeros((_NS, ke - kl, _CH), jnp.int32)], axis=1)
        hp = a[sl:sl + sh].reshape(_NS, ke, _CH)
        xp_ = a[sl + sh:].reshape(_NS, kx, _CH)
        pair = [lp, hp] if light == 0 else [hp, lp]
        return jnp.concatenate(pair, axis=0), xp_

    rowr2, rowx = parts(rowp)
    colr2, colx = parts(colp)
    xidx = jnp.stack([rowx, colx], axis=2)

    zeros_n1 = jnp.zeros((npad,), jnp.float32)
    zeros_n2 = jnp.zeros((npad, 128), jnp.float32)
    xp = jnp.zeros((npad, 128), jnp.float32).at[:n].set(x)

    degp = _deg_kernel(npad, ke)(colr, zeros_n1)
    h0, hs, db = _lin0_call(xp, lin0_w, lin0_b.reshape(1, 128), degp, npad)

    edge_scatter = _scatter_rows_kernel(npad, ke, kl, kx, light)
    h = h0
    for layer in range(NUM_LAYERS):
        beta = math.log(THETA / (layer + 1) + 1.0)
        p = edge_scatter(hs, rowr2, colr2, xidx, zeros_n2)
        h, hs = _layer_call(p, h, h0, db, conv_w[layer], beta, npad)

    return _pool_final_call(h, bat2, lin1_w, lin1_b.reshape(1, 128), npad)


# speedup vs baseline: 1.0965x; 1.0965x over previous
import functools
import math

import jax
import jax.numpy as jnp
from jax import lax
from jax.experimental import pallas as pl
from jax.experimental.pallas import tpu as pltpu
from jax.experimental.pallas import tpu_sc as plsc

NUM_LAYERS = 4
ALPHA = 0.1
THETA = 0.5
NUM_GRAPHS = 64

_NC = 2
_NS = 16
_NW = _NC * _NS
_CH = 128
_BT = 256


def _sc_mesh():
    return plsc.VectorSubcoreMesh(core_axis_name="c", subcore_axis_name="s")


def _deg_kernel(npad, kd):
    @functools.partial(
        pl.kernel,
        out_type=jax.ShapeDtypeStruct((_NC, npad), jnp.float32),
        mesh=_sc_mesh(),
        scratch_types=[
            pltpu.VMEM((kd, _CH), jnp.int32),
            pltpu.VMEM((_CH,), jnp.float32),
            pltpu.VMEM_SHARED((npad,), jnp.float32),
        ],
    )
    def k(col_hbm, zeros_hbm, deg_out, cidx, ones_v, deg_sh):
        cid = lax.axis_index("c")
        sid = lax.axis_index("s")
        wid = cid * _NS + sid
        rp = npad // _NS
        pltpu.sync_copy(zeros_hbm.at[pl.ds(sid * rp, rp)],
                        deg_sh.at[pl.ds(sid * rp, rp)])
        pltpu.sync_copy(col_hbm.at[wid], cidx)
        for j in range(_CH // 16):
            ones_v[pl.ds(j * 16, 16)] = jnp.ones((16,), jnp.float32)
        plsc.subcore_barrier()

        def dbody(j, c):
            pltpu.sync_copy(ones_v, deg_sh.at[cidx.at[j]], add=True)
            return c
        lax.fori_loop(0, kd, dbody, 0)

        plsc.subcore_barrier()
        pltpu.sync_copy(deg_sh.at[pl.ds(sid * rp, rp)],
                        deg_out.at[cid, pl.ds(sid * rp, rp)])

    return k


def _scatter_rows_kernel(mpad, k, k_light, kx, light_core):
    @functools.partial(
        pl.kernel,
        out_type=jax.ShapeDtypeStruct((_NC, mpad, 128), jnp.float32),
        mesh=_sc_mesh(),
        scratch_types=[
            pltpu.VMEM((k, _CH), jnp.int32),
            pltpu.VMEM((k, _CH), jnp.int32),
            pltpu.VMEM((2, _CH), jnp.int32),
            pltpu.VMEM((_CH, 128), jnp.float32),
            pltpu.VMEM_SHARED((mpad, 128), jnp.float32),
            pltpu.SemaphoreType.DMA,
        ],
    )
    def kfn(tab_hbm, ridx_hbm, cidx_hbm, xidx_hbm, zeros_hbm, out_hbm,
            ridx, cidx, xbuf, rows_v, agg_sh, sem):
        cid = lax.axis_index("c")
        sid = lax.axis_index("s")
        wid = cid * _NS + sid
        rp = mpad // _NS
        pltpu.sync_copy(zeros_hbm.at[pl.ds(sid * rp, rp)],
                        agg_sh.at[pl.ds(sid * rp, rp)])
        pltpu.sync_copy(ridx_hbm.at[wid], ridx)
        pltpu.sync_copy(cidx_hbm.at[wid], cidx)
        plsc.subcore_barrier()

        kk = jnp.where(cid == light_core, k_light, k)

        def body(j, c):
            pltpu.async_copy(tab_hbm.at[ridx.at[j]], rows_v, sem).wait()
            pltpu.sync_copy(rows_v, agg_sh.at[cidx.at[j]], add=True)
            return c
        lax.fori_loop(0, kk, body, 0)

        @pl.when(cid != light_core)
        def _():
            def xbody(j, c):
                pltpu.sync_copy(xidx_hbm.at[sid, j], xbuf)
                pltpu.async_copy(tab_hbm.at[xbuf.at[0]], rows_v, sem).wait()
                pltpu.sync_copy(rows_v, agg_sh.at[xbuf.at[1]], add=True)
                return c
            lax.fori_loop(0, kx, xbody, 0)

        plsc.subcore_barrier()
        pltpu.sync_copy(agg_sh.at[pl.ds(sid * rp, rp)],
                        out_hbm.at[cid, pl.ds(sid * rp, rp)])

    return kfn


def _lin0_call(xp, w, b, degp, npad):
    nblk = npad // _BT

    def body(x_ref, w_ref, b_ref, deg_ref, h_ref, hs_ref, db_ref):
        d = deg_ref[0, :] + deg_ref[1, :] + 1.0
        dinv = lax.rsqrt(d)
        h = jnp.maximum(
            jnp.dot(x_ref[...], w_ref[...],
                    preferred_element_type=jnp.float32) + b_ref[...], 0.0)
        db = jnp.broadcast_to(dinv[:, None], h.shape)
        h_ref[...] = h
        hs_ref[...] = h * db
        db_ref[...] = db

    o = jax.ShapeDtypeStruct((npad, 128), jnp.float32)
    return pl.pallas_call(
        body,
        grid=(nblk,),
        in_specs=[
            pl.BlockSpec((_BT, 128), lambda i: (i, 0)),
            pl.BlockSpec((128, 128), lambda i: (0, 0)),
            pl.BlockSpec((1, 128), lambda i: (0, 0)),
            pl.BlockSpec((2, _BT), lambda i: (0, i)),
        ],
        out_specs=[pl.BlockSpec((_BT, 128), lambda i: (i, 0))] * 3,
        out_shape=[o, o, o],
    )(xp, w, b, degp)


def _layer_call(p, h, x0, db, w, beta, npad):
    nblk = npad // _BT
    a1 = 1.0 - ALPHA
    b1 = 1.0 - beta

    def body(p_ref, h_ref, x0_ref, db_ref, w_ref, hn_ref, hs_ref):
        dbv = db_ref[...]
        s = p_ref[0] + p_ref[1]
        agg = dbv * s + dbv * dbv * h_ref[...]
        out = a1 * agg + ALPHA * x0_ref[...]
        m = jnp.dot(out, w_ref[...], preferred_element_type=jnp.float32)
        hn = jnp.maximum(b1 * out + beta * m, 0.0)
        hn_ref[...] = hn
        hs_ref[...] = hn * dbv

    o = jax.ShapeDtypeStruct((npad, 128), jnp.float32)
    return pl.pallas_call(
        body,
        grid=(nblk,),
        in_specs=[
            pl.BlockSpec((2, _BT, 128), lambda i: (0, i, 0)),
            pl.BlockSpec((_BT, 128), lambda i: (i, 0)),
            pl.BlockSpec((_BT, 128), lambda i: (i, 0)),
            pl.BlockSpec((_BT, 128), lambda i: (i, 0)),
            pl.BlockSpec((128, 128), lambda i: (0, 0)),
        ],
        out_specs=[pl.BlockSpec((_BT, 128), lambda i: (i, 0))] * 2,
        out_shape=[o, o],
    )(p, h, x0, db, w)


def _pool_final_call(h, bat2, w, b, npad):
    nblk = npad // _BT

    def body(bat_ref, h_ref, w_ref, b_ref, o_ref, psum, pcnt):
        i = pl.program_id(0)

        @pl.when(i == 0)
        def _():
            psum[...] = jnp.zeros_like(psum)
            pcnt[...] = jnp.zeros_like(pcnt)

        oh = jnp.equal(
            bat_ref[0][:, None],
            lax.broadcasted_iota(jnp.int32, (_BT, NUM_GRAPHS), 1)
        ).astype(jnp.float32)
        dn = (((0,), (0,)), ((), ()))
        hv = h_ref[...]
        psum[...] += lax.dot_general(oh, hv, dn,
                                     preferred_element_type=jnp.float32)
        pcnt[...] += lax.dot_general(oh, jnp.ones_like(hv), dn,
                                     preferred_element_type=jnp.float32)

        @pl.when(i == nblk - 1)
        def _():
            pooled = psum[...] / jnp.maximum(pcnt[...], 1.0)
            o_ref[...] = jnp.dot(pooled, w_ref[...],
                                 preferred_element_type=jnp.float32) \
                + b_ref[...]

    return pl.pallas_call(
        body,
        grid=(nblk,),
        in_specs=[
            pl.BlockSpec((1, _BT), lambda i: (0, i)),
            pl.BlockSpec((_BT, 128), lambda i: (i, 0)),
            pl.BlockSpec((128, 128), lambda i: (0, 0)),
            pl.BlockSpec((1, 128), lambda i: (0, 0)),
        ],
        out_specs=pl.BlockSpec((NUM_GRAPHS, 128), lambda i: (0, 0)),
        out_shape=jax.ShapeDtypeStruct((NUM_GRAPHS, 128), jnp.float32),
        scratch_shapes=[
            pltpu.VMEM((NUM_GRAPHS, 128), jnp.float32),
            pltpu.VMEM((NUM_GRAPHS, 128), jnp.float32),
        ],
    )(bat2, h, w, b)


def _ceil_to(v, m):
    return -(-v // m) * m


def kernel(x, edge_index, edge_attr, batch, lin0_w, lin0_b, conv_w,
           lin1_w, lin1_b):
    n = x.shape[0]
    e = edge_index.shape[1]
    npad = _ceil_to(n + 1, 2048)

    ke = _ceil_to(e, _NW * _CH) // (_NW * _CH)
    ep = ke * _NW * _CH

    row = edge_index[0]
    col = edge_index[1]
    colr = jnp.concatenate(
        [col, jnp.full((ep - e,), n, jnp.int32)]).reshape(_NW, ke, _CH)
    bat2 = jnp.concatenate(
        [batch, jnp.full((npad - n,), NUM_GRAPHS, jnp.int32)]).reshape(1, npad)

    light = 1
    kt = _ceil_to(e, _NS * _CH) // (_NS * _CH)
    kl = min(ke, max(0, round(kt * 0.37)))
    kx = max(0, kt - ke - kl)
    cap = _NS * (kl + ke + kx) * _CH
    rowp = jnp.concatenate([row, jnp.zeros((cap - e,), jnp.int32)])
    colp = jnp.concatenate([col, jnp.full((cap - e,), n, jnp.int32)])
    sl, sh = _NS * kl * _CH, _NS * ke * _CH

    def parts(a):
        lp = jnp.concatenate(
            [a[:sl].reshape(_NS, kl, _CH),
             jnp.zeros((_NS, ke - kl, _CH), jnp.int32)], axis=1)
        hp = a[sl:sl + sh].reshape(_NS, ke, _CH)
        xp_ = a[sl + sh:].reshape(_NS, kx, _CH)
        pair = [lp, hp] if light == 0 else [hp, lp]
        return jnp.concatenate(pair, axis=0), xp_

    rowr2, rowx = parts(rowp)
    colr2, colx = parts(colp)
    xidx = jnp.stack([rowx, colx], axis=2)

    zeros_n1 = jnp.zeros((npad,), jnp.float32)
    zeros_n2 = jnp.zeros((npad, 128), jnp.float32)
    xp = jnp.zeros((npad, 128), jnp.float32).at[:n].set(x)

    degp = _deg_kernel(npad, ke)(colr, zeros_n1)
    h0, hs, db = _lin0_call(xp, lin0_w, lin0_b.reshape(1, 128), degp, npad)

    edge_scatter = _scatter_rows_kernel(npad, ke, kl, kx, light)
    h = h0
    for layer in range(NUM_LAYERS):
        beta = math.log(THETA / (layer + 1) + 1.0)
        p = edge_scatter(hs, rowr2, colr2, xidx, zeros_n2)
        h, hs = _layer_call(p, h, h0, db, conv_w[layer], beta, npad)

    return _pool_final_call(h, bat2, lin1_w, lin1_b.reshape(1, 128), npad)
